# baseline (speedup 1.0000x reference)
import functools

import jax
import jax.numpy as jnp
from jax import lax
from jax.experimental import pallas as pl
from jax.experimental.pallas import tpu as pltpu
from jax.experimental.pallas import tpu_sc as plsc

L = 16
NC = 2
NS = 16
SLAB_PER_CORE = 1


def _leaky(v):
    return jnp.where(v >= 0, v, 0.01 * v)


def _mesh():
    return plsc.VectorSubcoreMesh(core_axis_name="c", subcore_axis_name="s")


def _sc_params():
    return pltpu.CompilerParams(needs_layout_passes=False,
                                use_tc_tiling_on_sc=False)


def _deg_body(NP, EB, dst_hbm, ew_hbm, out_hbm, dstw, eww, zbuf, dega):
    core = lax.axis_index("c")
    sid = lax.axis_index("s")
    half = EB // NC
    rt = half // NS
    stripe = NP // NS

    def zb(i, _):
        zbuf[pl.ds(i * L, L)] = jnp.zeros((L,), jnp.float32)
        return _
    lax.fori_loop(0, stripe // L, zb, None)
    pltpu.sync_copy(zbuf, dega.at[pl.ds(sid * stripe, stripe)])
    plsc.subcore_barrier()

    nwin = rt // dstw.shape[0]
    chb = dstw.shape[0]

    def win(wi, _):
        base = core * half + sid * rt + wi * chb
        pltpu.sync_copy(dst_hbm.at[pl.ds(base, chb)], dstw)
        pltpu.sync_copy(ew_hbm.at[pl.ds(base, chb)], eww)
        for b in range(chb):
            pltpu.sync_copy(eww.at[b], dega.at[dstw.at[b]], add=True)
        return _
    lax.fori_loop(0, nwin, win, None)

    plsc.subcore_barrier()
    pltpu.sync_copy(dega.at[pl.ds(sid * stripe, stripe)], zbuf)
    pltpu.sync_copy(zbuf,
                    out_hbm.at[pl.ds(core * NP + sid * stripe, stripe)])


def _deg_call(NP, EB, CHB, dst2d, ew2d):
    kfn = pl.kernel(
        functools.partial(_deg_body, NP, EB),
        out_type=jax.ShapeDtypeStruct((NC * NP,), jnp.float32),
        mesh=_mesh(),
        compiler_params=_sc_params(),
        scratch_types=[
            pltpu.VMEM((CHB, 128), jnp.int32),
            pltpu.VMEM((CHB, 128), jnp.float32),
            pltpu.VMEM((NP // NS,), jnp.float32),
            pltpu.VMEM_SHARED((NP,), jnp.float32),
        ],
    )
    return kfn(dst2d, ew2d)


def _norm_body(NP, EB, dinv_hbm, src_hbm, dst_hbm, ew_hbm, out_hbm,
               dinvv, srcw, dstw, eww, normw):
    core = lax.axis_index("c")
    sid = lax.axis_index("s")
    wid = sid * NC + core
    rt = EB // (NC * NS)

    pltpu.sync_copy(dinv_hbm, dinvv)

    chb = srcw.shape[0]
    nwin = rt // chb

    def win(wi, _):
        base = wid * rt + wi * chb
        pltpu.sync_copy(src_hbm.at[pl.ds(base, chb)], srcw)
        pltpu.sync_copy(dst_hbm.at[pl.ds(base, chb)], dstw)
        pltpu.sync_copy(ew_hbm.at[pl.ds(base, chb)], eww)

        def body(i, _):
            bb = i // 8
            off = (i % 8) * L
            sv = srcw[bb, pl.ds(off, L)]
            dv = dstw[bb, pl.ds(off, L)]
            ds = plsc.load_gather(dinvv, [sv])
            dd = plsc.load_gather(dinvv, [dv])
            normw[bb, pl.ds(off, L)] = ds * eww[bb, pl.ds(off, L)] * dd
            return _
        lax.fori_loop(0, chb * 8, body, None)

        pltpu.sync_copy(normw, out_hbm.at[pl.ds(base, chb)])
        return _
    lax.fori_loop(0, nwin, win, None)


def _norm_call(NP, EB, CHB, dinv, src2d, dst2d, ew2d):
    kfn = pl.kernel(
        functools.partial(_norm_body, NP, EB),
        out_type=jax.ShapeDtypeStruct((EB, 128), jnp.float32),
        mesh=_mesh(),
        compiler_params=_sc_params(),
        scratch_types=[
            pltpu.VMEM((NP,), jnp.float32),
            pltpu.VMEM((CHB, 128), jnp.int32),
            pltpu.VMEM((CHB, 128), jnp.int32),
            pltpu.VMEM((CHB, 128), jnp.float32),
            pltpu.VMEM((CHB, 128), jnp.float32),
        ],
    )
    return kfn(dinv, src2d, dst2d, ew2d)


def _agg_body(NP, EB, xs_hbm, src_hbm, dst_hbm, norm_hbm, np_hbm, out_hbm,
              srcw, dstw, normw, rows, zbuf, npass_s, acc, semg, sems):
    core = lax.axis_index("c")
    sid = lax.axis_index("s")
    rt = EB // NS
    stripe = NP // NS
    chb = srcw.shape[0]
    nwin = rt // chb

    pltpu.sync_copy(np_hbm, npass_s)
    npass = npass_s[pl.ds(0, L)][0]

    zchunk = zbuf.shape[0]
    nz = stripe // zchunk

    def one_pass(p, _):
        sg = p * NC + core

        def zb(i, _):
            zbuf[i, :] = jnp.zeros((L,), jnp.float32)
            return _
        lax.fori_loop(0, zchunk, zb, None)
        for q in range(nz):
            pltpu.sync_copy(
                zbuf, acc.at[pl.ds(sid * stripe + q * zchunk, zchunk)])
        plsc.subcore_barrier()

        def drain(par, sem):
            for b in range(chb):
                pltpu.make_async_copy(xs_hbm.at[sg, pl.ds(0, 128)],
                                      rows.at[par, b], sem).wait()

        def win(wi, _):
            par = lax.rem(wi, 2)

            @pl.when(wi >= 2)
            def _():
                drain(par, sems)

            base = sid * rt + wi * chb
            pltpu.sync_copy(src_hbm.at[pl.ds(base, chb)], srcw)
            pltpu.sync_copy(dst_hbm.at[pl.ds(base, chb)], dstw.at[par])
            pltpu.sync_copy(norm_hbm.at[pl.ds(base, chb)], normw)
            for b in range(chb):
                pltpu.async_copy(xs_hbm.at[sg].at[srcw.at[b]],
                                 rows.at[par, b], semg)
            drain(par, semg)

            def scale(g, _):
                for b in range(chb):
                    nv = normw[b, pl.ds(g * L, L)]
                    for l in range(L):
                        w = g * L + l
                        s = nv[l]
                        rows[par, b, w] = rows[par, b, w] * s
                return _
            lax.fori_loop(0, 128 // L, scale, None)

            for b in range(chb):
                pltpu.async_copy(rows.at[par, b], acc.at[dstw.at[par, b]],
                                 sems, add=True)
            return _
        lax.fori_loop(0, nwin, win, None)

        drain((nwin - 2) % 2, sems)
        drain((nwin - 1) % 2, sems)

        plsc.subcore_barrier()
        for q in range(nz):
            pltpu.sync_copy(
                acc.at[pl.ds(sid * stripe + q * zchunk, zchunk)], zbuf)
            pltpu.sync_copy(zbuf, out_hbm.at[
                pl.ds(sg * NP + sid * stripe + q * zchunk, zchunk)])
        plsc.subcore_barrier()
        return _
    lax.fori_loop(0, npass, one_pass, None)


SMAX = 16


def _agg_call(NP, EB, CHB, xs, src2d, dst2d, norm2d, npass):
    kfn = pl.kernel(
        functools.partial(_agg_body, NP, EB),
        out_type=jax.ShapeDtypeStruct((SMAX * NP, L), jnp.float32),
        mesh=_mesh(),
        compiler_params=_sc_params(),
        scratch_types=[
            pltpu.VMEM((CHB, 128), jnp.int32),
            pltpu.VMEM((2, CHB, 128), jnp.int32),
            pltpu.VMEM((CHB, 128), jnp.float32),
            pltpu.VMEM((2, CHB, 128, L), jnp.float32),
            pltpu.VMEM((784, L), jnp.float32),
            pltpu.VMEM((L,), jnp.int32),
            pltpu.VMEM_SHARED((NP, L), jnp.float32),
            pltpu.SemaphoreType.DMA,
            pltpu.SemaphoreType.DMA,
        ],
    )
    npa = jnp.full((L,), npass, dtype=jnp.int32)
    return kfn(xs, src2d, dst2d, norm2d, npa)


def _dinv_tc(dp_ref, o_ref):
    o_ref[...] = lax.rsqrt(1.0 + dp_ref[0] + dp_ref[1])


def _gcn_tc(acc_ref, x_ref, d2_ref, w_ref, b_ref, o_ref):
    a = acc_ref[...] + d2_ref[...] * x_ref[...]
    h = jnp.dot(a, w_ref[...], preferred_element_type=jnp.float32)
    o_ref[...] = _leaky(h + b_ref[...])


def _mlp_tc(f_ref, w0, b0, w1, b1, w2, b2, wo, bo, o_ref):
    y = _leaky(jnp.dot(f_ref[...], w0[...],
                       preferred_element_type=jnp.float32) + b0[...])
    y = _leaky(jnp.dot(y, w1[...], preferred_element_type=jnp.float32)
               + b1[...])
    y = _leaky(jnp.dot(y, w2[...], preferred_element_type=jnp.float32)
               + b2[...])
    o_ref[...] = jnp.dot(y, wo[...], preferred_element_type=jnp.float32) \
        + bo[...]


def _gcn_tc_call(NP, BN, FI, FO, accn, xp, dinvsq, W, b):
    grid = NP // BN
    return pl.pallas_call(
        _gcn_tc,
        grid=(grid,),
        in_specs=[
            pl.BlockSpec((BN, FI), lambda i: (i, 0)),
            pl.BlockSpec((BN, FI), lambda i: (i, 0)),
            pl.BlockSpec((BN, 1), lambda i: (i, 0)),
            pl.BlockSpec((FI, FO), lambda i: (0, 0)),
            pl.BlockSpec((1, FO), lambda i: (0, 0)),
        ],
        out_specs=pl.BlockSpec((BN, FO), lambda i: (i, 0)),
        out_shape=jax.ShapeDtypeStruct((NP, FO), jnp.float32),
    )(accn, xp, dinvsq, W, b)


def kernel(x, edge_index, edge_weight, W1, b1, W2, b2,
           Wf0, bf0, Wf1, bf1, Wf2, bf2, Wo, bo):
    N, F = x.shape
    E = edge_weight.shape[0]
    H = W1.shape[1]
    M = 22
    R = N // M

    NP = 50176
    EB = 6400
    EP = EB * 128
    F64 = 64
    S1 = F64 // L
    S2 = H // L

    src = edge_index[0]
    dst = edge_index[1]
    pad = EP - E
    src_p = jnp.concatenate([src, jnp.zeros((pad,), jnp.int32)])
    dst_p = jnp.concatenate(
        [dst, (jnp.arange(pad, dtype=jnp.int32) * 97) % N])
    ew_p = jnp.concatenate([edge_weight, jnp.zeros((pad,), jnp.float32)])
    src2d = src_p.reshape(EB, 128)
    dst2d = dst_p.reshape(EB, 128)
    ew2d = ew_p.reshape(EB, 128)

    degp = _deg_call(NP, EB, 40, dst2d, ew2d).reshape(NC, NP)
    dinv = pl.pallas_call(
        _dinv_tc,
        out_shape=jax.ShapeDtypeStruct((NP,), jnp.float32),
    )(degp)

    norm2d = _norm_call(NP, EB, 40, dinv, src2d, dst2d, ew2d)

    x64 = jnp.pad(x, ((0, NP - N), (0, F64 - F)))
    xs1 = jnp.pad(x64.reshape(NP, S1, L).transpose(1, 0, 2),
                  ((0, SMAX - S1), (0, 0), (0, 0)))
    acc1 = _agg_call(NP, EB, 8, xs1, src2d, dst2d, norm2d, S1 // NC)
    acc1n = acc1[:S1 * NP].reshape(S1, NP, L).transpose(1, 0, 2) \
        .reshape(NP, F64)

    dinvsq = (dinv * dinv).reshape(NP, 1)
    W1p = jnp.pad(W1, ((0, F64 - F), (0, 0)))
    h1 = _gcn_tc_call(NP, 1792, F64, H, acc1n, x64, dinvsq, W1p,
                      b1.reshape(1, H))

    h1s = h1.reshape(NP, S2, L).transpose(1, 0, 2)
    acc2 = _agg_call(NP, EB, 8, h1s, src2d, dst2d, norm2d, S2 // NC)
    acc2n = acc2.reshape(S2, NP, L).transpose(1, 0, 2).reshape(NP, H)

    h2 = _gcn_tc_call(NP, 1792, H, H, acc2n, h1, dinvsq, W2,
                      b2.reshape(1, H))

    cat = jnp.concatenate([h2[:N], x], axis=1)
    flat = cat.reshape(R, M * (H + F))
    RP = 2304
    KP = 7040
    flatp = jnp.pad(flat, ((0, RP - R), (0, KP - M * (H + F))))
    Wf0p = jnp.pad(Wf0, ((0, KP - M * (H + F)), (0, 0)))
    Wop = jnp.pad(Wo, ((0, 0), (0, 128 - Wo.shape[1])))
    bop = jnp.pad(bo, ((0, 128 - bo.shape[0]),))

    BR = 384
    out = pl.pallas_call(
        _mlp_tc,
        grid=(RP // BR,),
        in_specs=[
            pl.BlockSpec((BR, KP), lambda i: (i, 0)),
            pl.BlockSpec((KP, H), lambda i: (0, 0)),
            pl.BlockSpec((1, H), lambda i: (0, 0)),
            pl.BlockSpec((H, H), lambda i: (0, 0)),
            pl.BlockSpec((1, H), lambda i: (0, 0)),
            pl.BlockSpec((H, H), lambda i: (0, 0)),
            pl.BlockSpec((1, H), lambda i: (0, 0)),
            pl.BlockSpec((H, 128), lambda i: (0, 0)),
            pl.BlockSpec((1, 128), lambda i: (0, 0)),
        ],
        out_specs=pl.BlockSpec((BR, 128), lambda i: (i, 0)),
        out_shape=jax.ShapeDtypeStruct((RP, 128), jnp.float32),
    )(flatp, Wf0p, bf0.reshape(1, H), Wf1, bf1.reshape(1, H),
      Wf2, bf2.reshape(1, H), Wop, bop.reshape(1, 128))

    return out[:R, :4]


# speedup vs baseline: 1.6820x; 1.6820x over previous
import functools

import jax
import jax.numpy as jnp
from jax import lax
from jax.experimental import pallas as pl
from jax.experimental.pallas import tpu as pltpu
from jax.experimental.pallas import tpu_sc as plsc

L = 16
NC = 2
NS = 16
SLAB_PER_CORE = 1


def _leaky(v):
    return jnp.where(v >= 0, v, 0.01 * v)


def _mesh():
    return plsc.VectorSubcoreMesh(core_axis_name="c", subcore_axis_name="s")


def _sc_params():
    return pltpu.CompilerParams(needs_layout_passes=False,
                                use_tc_tiling_on_sc=False)


def _deg_body(NP, EB, dst_hbm, ew_hbm, out_hbm, dstw, eww, zbuf, dega):
    core = lax.axis_index("c")
    sid = lax.axis_index("s")
    half = EB // NC
    rt = half // NS
    stripe = NP // NS

    def zb(i, _):
        zbuf[pl.ds(i * L, L)] = jnp.zeros((L,), jnp.float32)
        return _
    lax.fori_loop(0, stripe // L, zb, None)
    pltpu.sync_copy(zbuf, dega.at[pl.ds(sid * stripe, stripe)])
    plsc.subcore_barrier()

    nwin = rt // dstw.shape[0]
    chb = dstw.shape[0]

    def win(wi, _):
        base = core * half + sid * rt + wi * chb
        pltpu.sync_copy(dst_hbm.at[pl.ds(base, chb)], dstw)
        pltpu.sync_copy(ew_hbm.at[pl.ds(base, chb)], eww)
        for b in range(chb):
            pltpu.sync_copy(eww.at[b], dega.at[dstw.at[b]], add=True)
        return _
    lax.fori_loop(0, nwin, win, None)

    plsc.subcore_barrier()
    pltpu.sync_copy(dega.at[pl.ds(sid * stripe, stripe)], zbuf)
    pltpu.sync_copy(zbuf,
                    out_hbm.at[pl.ds(core * NP + sid * stripe, stripe)])


def _deg_call(NP, EB, CHB, dst2d, ew2d):
    kfn = pl.kernel(
        functools.partial(_deg_body, NP, EB),
        out_type=jax.ShapeDtypeStruct((NC * NP,), jnp.float32),
        mesh=_mesh(),
        compiler_params=_sc_params(),
        scratch_types=[
            pltpu.VMEM((CHB, 128), jnp.int32),
            pltpu.VMEM((CHB, 128), jnp.float32),
            pltpu.VMEM((NP // NS,), jnp.float32),
            pltpu.VMEM_SHARED((NP,), jnp.float32),
        ],
    )
    return kfn(dst2d, ew2d)


def _norm_body(NP, EB, dinv_hbm, src_hbm, dst_hbm, ew_hbm, out_hbm,
               dinvv, srcw, dstw, eww, normw):
    core = lax.axis_index("c")
    sid = lax.axis_index("s")
    wid = sid * NC + core
    rt = EB // (NC * NS)

    pltpu.sync_copy(dinv_hbm, dinvv)

    chb = srcw.shape[0]
    nwin = rt // chb

    def win(wi, _):
        base = wid * rt + wi * chb
        pltpu.sync_copy(src_hbm.at[pl.ds(base, chb)], srcw)
        pltpu.sync_copy(dst_hbm.at[pl.ds(base, chb)], dstw)
        pltpu.sync_copy(ew_hbm.at[pl.ds(base, chb)], eww)

        def body(i, _):
            bb = i // 8
            off = (i % 8) * L
            sv = srcw[bb, pl.ds(off, L)]
            dv = dstw[bb, pl.ds(off, L)]
            ds = plsc.load_gather(dinvv, [sv])
            dd = plsc.load_gather(dinvv, [dv])
            normw[bb, pl.ds(off, L)] = ds * eww[bb, pl.ds(off, L)] * dd
            return _
        lax.fori_loop(0, chb * 8, body, None)

        pltpu.sync_copy(normw, out_hbm.at[pl.ds(base, chb)])
        return _
    lax.fori_loop(0, nwin, win, None)


def _norm_call(NP, EB, CHB, dinv, src2d, dst2d, ew2d):
    kfn = pl.kernel(
        functools.partial(_norm_body, NP, EB),
        out_type=jax.ShapeDtypeStruct((EB, 128), jnp.float32),
        mesh=_mesh(),
        compiler_params=_sc_params(),
        scratch_types=[
            pltpu.VMEM((NP,), jnp.float32),
            pltpu.VMEM((CHB, 128), jnp.int32),
            pltpu.VMEM((CHB, 128), jnp.int32),
            pltpu.VMEM((CHB, 128), jnp.float32),
            pltpu.VMEM((CHB, 128), jnp.float32),
        ],
    )
    return kfn(dinv, src2d, dst2d, ew2d)


def _agg_body(NP, EB, S, CH, xs_hbm, src_hbm, dst_hbm, norm_hbm, out_hbm,
              srcw, dstw, normw, rows, zbuf, acc, semg, sems):
    core = lax.axis_index("c")
    sid = lax.axis_index("s")
    rt = (EB // NS) * 128
    stripe = NP // NS
    nwin = rt // CH
    npass = S // NC

    zchunk = zbuf.shape[0]
    nz = stripe // zchunk

    for p in range(npass):
        sg = p * NC + core

        def zb(i, _):
            zbuf[i, :] = jnp.zeros((L,), jnp.float32)
            return _
        lax.fori_loop(0, zchunk, zb, None)
        for q in range(nz):
            pltpu.sync_copy(
                zbuf, acc.at[pl.ds(sid * stripe + q * zchunk, zchunk)])
        plsc.subcore_barrier()

        def drain_scatter(par):
            pltpu.make_async_copy(xs_hbm.at[sg, pl.ds(0, CH)],
                                  rows.at[par], sems).wait()

        def one_window(wi, par):
            base = sid * rt + wi * CH
            pltpu.async_copy(src_hbm.at[pl.ds(base, CH)], srcw, semg)
            pltpu.async_copy(dst_hbm.at[pl.ds(base, CH)], dstw.at[par],
                             semg)
            pltpu.async_copy(norm_hbm.at[pl.ds(base, CH)], normw, semg)
            pltpu.make_async_copy(src_hbm.at[pl.ds(base, CH)], srcw,
                                  semg).wait()
            pltpu.make_async_copy(dst_hbm.at[pl.ds(base, CH)],
                                  dstw.at[par], semg).wait()
            pltpu.make_async_copy(norm_hbm.at[pl.ds(base, CH)], normw,
                                  semg).wait()
            pltpu.async_copy(xs_hbm.at[sg].at[srcw], rows.at[par], semg)
            pltpu.make_async_copy(xs_hbm.at[sg, pl.ds(0, CH)],
                                  rows.at[par], semg).wait()

            def scale(g, _):
                nv = normw[pl.ds(g * L, L)]
                for l in range(L):
                    w = g * L + l
                    rows[par, w] = rows[par, w] * nv[l]
                return _
            lax.fori_loop(0, CH // L, scale, None)

            pltpu.async_copy(rows.at[par], acc.at[dstw.at[par]], sems,
                             add=True)

        def win2(wi2, _):
            for par in range(2):
                @pl.when(wi2 >= 1)
                def _():
                    drain_scatter(par)
                one_window(wi2 * 2 + par, par)
            return _
        lax.fori_loop(0, nwin // 2, win2, None)
        drain_scatter(0)
        drain_scatter(1)

        plsc.subcore_barrier()
        for q in range(nz):
            pltpu.sync_copy(
                acc.at[pl.ds(sid * stripe + q * zchunk, zchunk)], zbuf)
            pltpu.sync_copy(zbuf, out_hbm.at[
                pl.ds(sg * NP + sid * stripe + q * zchunk, zchunk)])
        plsc.subcore_barrier()


def _agg_call(NP, EB, S, CH, xs, src1d, dst1d, norm1d):
    kfn = pl.kernel(
        functools.partial(_agg_body, NP, EB, S, CH),
        out_type=jax.ShapeDtypeStruct((S * NP, L), jnp.float32),
        mesh=_mesh(),
        compiler_params=_sc_params(),
        scratch_types=[
            pltpu.VMEM((CH,), jnp.int32),
            pltpu.VMEM((2, CH), jnp.int32),
            pltpu.VMEM((CH,), jnp.float32),
            pltpu.VMEM((2, CH, L), jnp.float32),
            pltpu.VMEM((784, L), jnp.float32),
            pltpu.VMEM_SHARED((NP, L), jnp.float32),
            pltpu.SemaphoreType.DMA,
            pltpu.SemaphoreType.DMA,
        ],
    )
    return kfn(xs, src1d, dst1d, norm1d)


def _dinv_tc(dp_ref, o_ref):
    o_ref[...] = lax.rsqrt(1.0 + dp_ref[0] + dp_ref[1])


def _gcn_tc(acc_ref, x_ref, d2_ref, w_ref, b_ref, o_ref):
    a = acc_ref[...] + d2_ref[...] * x_ref[...]
    h = jnp.dot(a, w_ref[...], preferred_element_type=jnp.float32)
    o_ref[...] = _leaky(h + b_ref[...])


def _mlp_tc(f_ref, w0, b0, w1, b1, w2, b2, wo, bo, o_ref):
    y = _leaky(jnp.dot(f_ref[...], w0[...],
                       preferred_element_type=jnp.float32) + b0[...])
    y = _leaky(jnp.dot(y, w1[...], preferred_element_type=jnp.float32)
               + b1[...])
    y = _leaky(jnp.dot(y, w2[...], preferred_element_type=jnp.float32)
               + b2[...])
    o_ref[...] = jnp.dot(y, wo[...], preferred_element_type=jnp.float32) \
        + bo[...]


def _gcn_tc_call(NP, BN, FI, FO, accn, xp, dinvsq, W, b):
    grid = NP // BN
    return pl.pallas_call(
        _gcn_tc,
        grid=(grid,),
        in_specs=[
            pl.BlockSpec((BN, FI), lambda i: (i, 0)),
            pl.BlockSpec((BN, FI), lambda i: (i, 0)),
            pl.BlockSpec((BN, 1), lambda i: (i, 0)),
            pl.BlockSpec((FI, FO), lambda i: (0, 0)),
            pl.BlockSpec((1, FO), lambda i: (0, 0)),
        ],
        out_specs=pl.BlockSpec((BN, FO), lambda i: (i, 0)),
        out_shape=jax.ShapeDtypeStruct((NP, FO), jnp.float32),
    )(accn, xp, dinvsq, W, b)


def kernel(x, edge_index, edge_weight, W1, b1, W2, b2,
           Wf0, bf0, Wf1, bf1, Wf2, bf2, Wo, bo):
    N, F = x.shape
    E = edge_weight.shape[0]
    H = W1.shape[1]
    M = 22
    R = N // M

    NP = 50176
    EB = 6400
    EP = EB * 128
    F64 = 64
    S1 = F64 // L
    S2 = H // L

    src = edge_index[0]
    dst = edge_index[1]
    pad = EP - E
    src_p = jnp.concatenate([src, jnp.zeros((pad,), jnp.int32)])
    dst_p = jnp.concatenate(
        [dst, (jnp.arange(pad, dtype=jnp.int32) * 97) % N])
    ew_p = jnp.concatenate([edge_weight, jnp.zeros((pad,), jnp.float32)])
    src2d = src_p.reshape(EB, 128)
    dst2d = dst_p.reshape(EB, 128)
    ew2d = ew_p.reshape(EB, 128)

    degp = _deg_call(NP, EB, 40, dst2d, ew2d).reshape(NC, NP)
    dinv = pl.pallas_call(
        _dinv_tc,
        out_shape=jax.ShapeDtypeStruct((NP,), jnp.float32),
    )(degp)

    norm2d = _norm_call(NP, EB, 40, dinv, src2d, dst2d, ew2d)

    x64 = jnp.pad(x, ((0, NP - N), (0, F64 - F)))
    xs1 = x64.reshape(NP, S1, L).transpose(1, 0, 2)
    norm1d = norm2d.reshape(EP)
    acc1 = _agg_call(NP, EB, S1, 1024, xs1, src_p, dst_p, norm1d)
    acc1n = acc1.reshape(S1, NP, L).transpose(1, 0, 2).reshape(NP, F64)

    dinvsq = (dinv * dinv).reshape(NP, 1)
    W1p = jnp.pad(W1, ((0, F64 - F), (0, 0)))
    h1 = _gcn_tc_call(NP, 1792, F64, H, acc1n, x64, dinvsq, W1p,
                      b1.reshape(1, H))

    h1s = h1.reshape(NP, S2, L).transpose(1, 0, 2)
    acc2 = _agg_call(NP, EB, S2, 1024, h1s, src_p, dst_p, norm1d)
    acc2n = acc2.reshape(S2, NP, L).transpose(1, 0, 2).reshape(NP, H)

    h2 = _gcn_tc_call(NP, 1792, H, H, acc2n, h1, dinvsq, W2,
                      b2.reshape(1, H))

    cat = jnp.concatenate([h2[:N], x], axis=1)
    flat = cat.reshape(R, M * (H + F))
    RP = 2304
    KP = 7040
    flatp = jnp.pad(flat, ((0, RP - R), (0, KP - M * (H + F))))
    Wf0p = jnp.pad(Wf0, ((0, KP - M * (H + F)), (0, 0)))
    Wop = jnp.pad(Wo, ((0, 0), (0, 128 - Wo.shape[1])))
    bop = jnp.pad(bo, ((0, 128 - bo.shape[0]),))

    BR = 384
    out = pl.pallas_call(
        _mlp_tc,
        grid=(RP // BR,),
        in_specs=[
            pl.BlockSpec((BR, KP), lambda i: (i, 0)),
            pl.BlockSpec((KP, H), lambda i: (0, 0)),
            pl.BlockSpec((1, H), lambda i: (0, 0)),
            pl.BlockSpec((H, H), lambda i: (0, 0)),
            pl.BlockSpec((1, H), lambda i: (0, 0)),
            pl.BlockSpec((H, H), lambda i: (0, 0)),
            pl.BlockSpec((1, H), lambda i: (0, 0)),
            pl.BlockSpec((H, 128), lambda i: (0, 0)),
            pl.BlockSpec((1, 128), lambda i: (0, 0)),
        ],
        out_specs=pl.BlockSpec((BR, 128), lambda i: (i, 0)),
        out_shape=jax.ShapeDtypeStruct((RP, 128), jnp.float32),
    )(flatp, Wf0p, bf0.reshape(1, H), Wf1, bf1.reshape(1, H),
      Wf2, bf2.reshape(1, H), Wop, bop.reshape(1, 128))

    return out[:R, :4]


# speedup vs baseline: 1.9874x; 1.1816x over previous
import functools

import jax
import jax.numpy as jnp
from jax import lax
from jax.experimental import pallas as pl
from jax.experimental.pallas import tpu as pltpu
from jax.experimental.pallas import tpu_sc as plsc

L = 16
NC = 2
NS = 16
SLAB_PER_CORE = 1


def _leaky(v):
    return jnp.where(v >= 0, v, 0.01 * v)


def _mesh():
    return plsc.VectorSubcoreMesh(core_axis_name="c", subcore_axis_name="s")


def _sc_params():
    return pltpu.CompilerParams(needs_layout_passes=False,
                                use_tc_tiling_on_sc=False)


def _deg_body(NP, EB, dst_hbm, ew_hbm, out_hbm, dstw, eww, zbuf, dega):
    core = lax.axis_index("c")
    sid = lax.axis_index("s")
    half = EB // NC
    rt = half // NS
    stripe = NP // NS

    def zb(i, _):
        zbuf[pl.ds(i * L, L)] = jnp.zeros((L,), jnp.float32)
        return _
    lax.fori_loop(0, stripe // L, zb, None)
    pltpu.sync_copy(zbuf, dega.at[pl.ds(sid * stripe, stripe)])
    plsc.subcore_barrier()

    nwin = rt // dstw.shape[0]
    chb = dstw.shape[0]

    def win(wi, _):
        base = core * half + sid * rt + wi * chb
        pltpu.sync_copy(dst_hbm.at[pl.ds(base, chb)], dstw)
        pltpu.sync_copy(ew_hbm.at[pl.ds(base, chb)], eww)
        for b in range(chb):
            pltpu.sync_copy(eww.at[b], dega.at[dstw.at[b]], add=True)
        return _
    lax.fori_loop(0, nwin, win, None)

    plsc.subcore_barrier()
    pltpu.sync_copy(dega.at[pl.ds(sid * stripe, stripe)], zbuf)
    pltpu.sync_copy(zbuf,
                    out_hbm.at[pl.ds(core * NP + sid * stripe, stripe)])


def _deg_call(NP, EB, CHB, dst2d, ew2d):
    kfn = pl.kernel(
        functools.partial(_deg_body, NP, EB),
        out_type=jax.ShapeDtypeStruct((NC * NP,), jnp.float32),
        mesh=_mesh(),
        compiler_params=_sc_params(),
        scratch_types=[
            pltpu.VMEM((CHB, 128), jnp.int32),
            pltpu.VMEM((CHB, 128), jnp.float32),
            pltpu.VMEM((NP // NS,), jnp.float32),
            pltpu.VMEM_SHARED((NP,), jnp.float32),
        ],
    )
    return kfn(dst2d, ew2d)


def _norm_body(NP, EB, dinv_hbm, src_hbm, dst_hbm, ew_hbm, out_hbm,
               dinvv, srcw, dstw, eww, normw):
    core = lax.axis_index("c")
    sid = lax.axis_index("s")
    wid = sid * NC + core
    rt = EB // (NC * NS)

    pltpu.sync_copy(dinv_hbm, dinvv)

    chb = srcw.shape[0]
    nwin = rt // chb

    def win(wi, _):
        base = wid * rt + wi * chb
        pltpu.sync_copy(src_hbm.at[pl.ds(base, chb)], srcw)
        pltpu.sync_copy(dst_hbm.at[pl.ds(base, chb)], dstw)
        pltpu.sync_copy(ew_hbm.at[pl.ds(base, chb)], eww)

        def body(i, _):
            bb = i // 8
            off = (i % 8) * L
            sv = srcw[bb, pl.ds(off, L)]
            dv = dstw[bb, pl.ds(off, L)]
            ds = plsc.load_gather(dinvv, [sv])
            dd = plsc.load_gather(dinvv, [dv])
            normw[bb, pl.ds(off, L)] = ds * eww[bb, pl.ds(off, L)] * dd
            return _
        lax.fori_loop(0, chb * 8, body, None)

        pltpu.sync_copy(normw, out_hbm.at[pl.ds(base, chb)])
        return _
    lax.fori_loop(0, nwin, win, None)


def _norm_call(NP, EB, CHB, dinv, src2d, dst2d, ew2d):
    kfn = pl.kernel(
        functools.partial(_norm_body, NP, EB),
        out_type=jax.ShapeDtypeStruct((EB, 128), jnp.float32),
        mesh=_mesh(),
        compiler_params=_sc_params(),
        scratch_types=[
            pltpu.VMEM((NP,), jnp.float32),
            pltpu.VMEM((CHB, 128), jnp.int32),
            pltpu.VMEM((CHB, 128), jnp.int32),
            pltpu.VMEM((CHB, 128), jnp.float32),
            pltpu.VMEM((CHB, 128), jnp.float32),
        ],
    )
    return kfn(dinv, src2d, dst2d, ew2d)


def _agg_body(NP, EB, S, CH, xs_hbm, src_hbm, dst_hbm, norm_hbm, out_hbm,
              srcw, dstw, normw, rows, zbuf, acc, semg, sems, semt):
    core = lax.axis_index("c")
    sid = lax.axis_index("s")
    rt = (EB // NS) * 128
    stripe = NP // NS
    nwin = rt // CH
    npass = S // NC

    zchunk = zbuf.shape[0]
    nz = stripe // zchunk

    for p in range(npass):
        sg = p * NC + core

        def zb(i, _):
            zbuf[i, :] = jnp.zeros((L,), jnp.float32)
            return _
        lax.fori_loop(0, zchunk, zb, None)
        for q in range(nz):
            pltpu.sync_copy(
                zbuf, acc.at[pl.ds(sid * stripe + q * zchunk, zchunk)])
        plsc.subcore_barrier()

        def drain_scatter(par):
            pltpu.make_async_copy(xs_hbm.at[sg, pl.ds(0, CH)],
                                  rows.at[par], sems).wait()

        def stage_fire(wi, par):
            base = sid * rt + wi * CH
            pltpu.async_copy(src_hbm.at[pl.ds(base, CH)], srcw.at[par],
                             semt)
            pltpu.async_copy(dst_hbm.at[pl.ds(base, CH)], dstw.at[par],
                             semt)
            pltpu.async_copy(norm_hbm.at[pl.ds(base, CH)], normw.at[par],
                             semt)
            pltpu.make_async_copy(src_hbm.at[pl.ds(base, CH)],
                                  srcw.at[par], semt).wait()
            pltpu.make_async_copy(dst_hbm.at[pl.ds(base, CH)],
                                  dstw.at[par], semt).wait()
            pltpu.make_async_copy(norm_hbm.at[pl.ds(base, CH)],
                                  normw.at[par], semt).wait()
            pltpu.async_copy(xs_hbm.at[sg].at[srcw.at[par]], rows.at[par],
                             semg)

        stage_fire(0, 0)

        def win2(wi2, _):
            for par in range(2):
                wi = wi2 * 2 + par

                @pl.when(wi >= 1)
                def _():
                    drain_scatter(1 - par)

                @pl.when(wi + 1 < nwin)
                def _():
                    stage_fire(wi + 1, 1 - par)

                pltpu.make_async_copy(xs_hbm.at[sg, pl.ds(0, CH)],
                                      rows.at[par], semg).wait()

                def scale(g, _):
                    nv = normw[par, pl.ds(g * L, L)]
                    for l in range(L):
                        w = g * L + l
                        rows[par, w] = rows[par, w] * nv[l]
                    return _
                lax.fori_loop(0, CH // L, scale, None)

                pltpu.async_copy(rows.at[par], acc.at[dstw.at[par]], sems,
                                 add=True)
            return _
        lax.fori_loop(0, nwin // 2, win2, None)
        drain_scatter((nwin - 1) % 2)

        plsc.subcore_barrier()
        for q in range(nz):
            pltpu.sync_copy(
                acc.at[pl.ds(sid * stripe + q * zchunk, zchunk)], zbuf)
            pltpu.sync_copy(zbuf, out_hbm.at[
                pl.ds(sg * NP + sid * stripe + q * zchunk, zchunk)])
        plsc.subcore_barrier()


def _agg_call(NP, EB, S, CH, xs, src1d, dst1d, norm1d):
    kfn = pl.kernel(
        functools.partial(_agg_body, NP, EB, S, CH),
        out_type=jax.ShapeDtypeStruct((S * NP, L), jnp.float32),
        mesh=_mesh(),
        compiler_params=_sc_params(),
        scratch_types=[
            pltpu.VMEM((2, CH), jnp.int32),
            pltpu.VMEM((2, CH), jnp.int32),
            pltpu.VMEM((2, CH), jnp.float32),
            pltpu.VMEM((2, CH, L), jnp.float32),
            pltpu.VMEM((784, L), jnp.float32),
            pltpu.VMEM_SHARED((NP, L), jnp.float32),
            pltpu.SemaphoreType.DMA,
            pltpu.SemaphoreType.DMA,
            pltpu.SemaphoreType.DMA,
        ],
    )
    return kfn(xs, src1d, dst1d, norm1d)


def _dinv_tc(dp_ref, o_ref):
    o_ref[...] = lax.rsqrt(1.0 + dp_ref[0] + dp_ref[1])


def _gcn_tc(acc_ref, x_ref, d2_ref, w_ref, b_ref, o_ref):
    a = acc_ref[...] + d2_ref[...] * x_ref[...]
    h = jnp.dot(a, w_ref[...], preferred_element_type=jnp.float32)
    o_ref[...] = _leaky(h + b_ref[...])


def _mlp_tc(f_ref, w0, b0, w1, b1, w2, b2, wo, bo, o_ref):
    y = _leaky(jnp.dot(f_ref[...], w0[...],
                       preferred_element_type=jnp.float32) + b0[...])
    y = _leaky(jnp.dot(y, w1[...], preferred_element_type=jnp.float32)
               + b1[...])
    y = _leaky(jnp.dot(y, w2[...], preferred_element_type=jnp.float32)
               + b2[...])
    o_ref[...] = jnp.dot(y, wo[...], preferred_element_type=jnp.float32) \
        + bo[...]


def _gcn_tc_call(NP, BN, FI, FO, accn, xp, dinvsq, W, b):
    grid = NP // BN
    return pl.pallas_call(
        _gcn_tc,
        grid=(grid,),
        in_specs=[
            pl.BlockSpec((BN, FI), lambda i: (i, 0)),
            pl.BlockSpec((BN, FI), lambda i: (i, 0)),
            pl.BlockSpec((BN, 1), lambda i: (i, 0)),
            pl.BlockSpec((FI, FO), lambda i: (0, 0)),
            pl.BlockSpec((1, FO), lambda i: (0, 0)),
        ],
        out_specs=pl.BlockSpec((BN, FO), lambda i: (i, 0)),
        out_shape=jax.ShapeDtypeStruct((NP, FO), jnp.float32),
    )(accn, xp, dinvsq, W, b)


def kernel(x, edge_index, edge_weight, W1, b1, W2, b2,
           Wf0, bf0, Wf1, bf1, Wf2, bf2, Wo, bo):
    N, F = x.shape
    E = edge_weight.shape[0]
    H = W1.shape[1]
    M = 22
    R = N // M

    NP = 50176
    EB = 6400
    EP = EB * 128
    F64 = 64
    S1 = F64 // L
    S2 = H // L

    src = edge_index[0]
    dst = edge_index[1]
    pad = EP - E
    src_p = jnp.concatenate([src, jnp.zeros((pad,), jnp.int32)])
    dst_p = jnp.concatenate(
        [dst, (jnp.arange(pad, dtype=jnp.int32) * 97) % N])
    ew_p = jnp.concatenate([edge_weight, jnp.zeros((pad,), jnp.float32)])
    src2d = src_p.reshape(EB, 128)
    dst2d = dst_p.reshape(EB, 128)
    ew2d = ew_p.reshape(EB, 128)

    degp = _deg_call(NP, EB, 40, dst2d, ew2d).reshape(NC, NP)
    dinv = pl.pallas_call(
        _dinv_tc,
        out_shape=jax.ShapeDtypeStruct((NP,), jnp.float32),
    )(degp)

    norm2d = _norm_call(NP, EB, 40, dinv, src2d, dst2d, ew2d)

    x64 = jnp.pad(x, ((0, NP - N), (0, F64 - F)))
    xs1 = x64.reshape(NP, S1, L).transpose(1, 0, 2)
    norm1d = norm2d.reshape(EP)
    acc1 = _agg_call(NP, EB, S1, 1024, xs1, src_p, dst_p, norm1d)
    acc1n = acc1.reshape(S1, NP, L).transpose(1, 0, 2).reshape(NP, F64)

    dinvsq = (dinv * dinv).reshape(NP, 1)
    W1p = jnp.pad(W1, ((0, F64 - F), (0, 0)))
    h1 = _gcn_tc_call(NP, 1792, F64, H, acc1n, x64, dinvsq, W1p,
                      b1.reshape(1, H))

    h1s = h1.reshape(NP, S2, L).transpose(1, 0, 2)
    acc2 = _agg_call(NP, EB, S2, 1024, h1s, src_p, dst_p, norm1d)
    acc2n = acc2.reshape(S2, NP, L).transpose(1, 0, 2).reshape(NP, H)

    h2 = _gcn_tc_call(NP, 1792, H, H, acc2n, h1, dinvsq, W2,
                      b2.reshape(1, H))

    cat = jnp.concatenate([h2[:N], x], axis=1)
    flat = cat.reshape(R, M * (H + F))
    RP = 2304
    KP = 7040
    flatp = jnp.pad(flat, ((0, RP - R), (0, KP - M * (H + F))))
    Wf0p = jnp.pad(Wf0, ((0, KP - M * (H + F)), (0, 0)))
    Wop = jnp.pad(Wo, ((0, 0), (0, 128 - Wo.shape[1])))
    bop = jnp.pad(bo, ((0, 128 - bo.shape[0]),))

    BR = 384
    out = pl.pallas_call(
        _mlp_tc,
        grid=(RP // BR,),
        in_specs=[
            pl.BlockSpec((BR, KP), lambda i: (i, 0)),
            pl.BlockSpec((KP, H), lambda i: (0, 0)),
            pl.BlockSpec((1, H), lambda i: (0, 0)),
            pl.BlockSpec((H, H), lambda i: (0, 0)),
            pl.BlockSpec((1, H), lambda i: (0, 0)),
            pl.BlockSpec((H, H), lambda i: (0, 0)),
            pl.BlockSpec((1, H), lambda i: (0, 0)),
            pl.BlockSpec((H, 128), lambda i: (0, 0)),
            pl.BlockSpec((1, 128), lambda i: (0, 0)),
        ],
        out_specs=pl.BlockSpec((BR, 128), lambda i: (i, 0)),
        out_shape=jax.ShapeDtypeStruct((RP, 128), jnp.float32),
    )(flatp, Wf0p, bf0.reshape(1, H), Wf1, bf1.reshape(1, H),
      Wf2, bf2.reshape(1, H), Wop, bop.reshape(1, 128))

    return out[:R, :4]
